# baseline (speedup 1.0000x reference)
import functools

import jax
import jax.numpy as jnp
from jax import lax
from jax.experimental import pallas as pl
from jax.experimental.pallas import tpu as pltpu
from jax.experimental.pallas import tpu_sc as plsc

XP = 100352
NB = XP // 128
KTOP = 100
NWORK = 32
CH = 64
CAND_ROWS = 832
SEL = 128


def _order_u32(x):
    u = lax.bitcast_convert_type(x, jnp.uint32)
    neg = u >= jnp.uint32(0x80000000)
    return jnp.where(neg, jnp.uint32(0) - u, u ^ jnp.uint32(0x80000000))


def _tc_body(nx, q_ref, it_ref, keys_ref, bm_ref):
    logits = jnp.dot(q_ref[...], it_ref[...],
                     preferred_element_type=jnp.float32)
    col = lax.broadcasted_iota(jnp.int32, logits.shape, 1)
    logits = jnp.where(col < nx, logits, jnp.float32(-1e30))
    keys_ref[...] = _order_u32(logits)
    r = logits.reshape(logits.shape[0], NB, 128)
    bm_ref[...] = _order_u32(jnp.max(r, axis=2))


def _sc_topk(keys2d, bm, vals_ref, gidx_ref,
             cand_buf, bm_v, cand_idx, cand_gid, hist, sel_o, sel_gi,
             stage_v, stage_i, smem, sem):
    nc = 2
    wid = lax.axis_index("s") * nc + lax.axis_index("c")
    lanes = lax.iota(jnp.int32, 16)
    ones = jnp.ones((16,), jnp.int32)
    rpw = 1024 // NWORK

    def popc(mask):
        return plsc.all_reduce_population_count(mask)[0]

    def hist_scan(kth):
        def cond(c):
            b, cum, last = c
            return jnp.logical_and(cum < kth, b >= 0)

        def body(c):
            b, cum, last = c
            t = jnp.sum(hist[pl.ds(b * 16, 16)])
            return (b - 1, cum + t, t)

        b, cum, last = lax.while_loop(cond, body, (jnp.int32(255),
                                                   jnp.int32(0),
                                                   jnp.int32(0)))
        return b + 1, cum - last, last

    def zero_hist():
        zv = jnp.zeros((16,), jnp.int32)

        def zh(i, _):
            for s in range(8):
                hist[pl.ds(i * 128 + s * 16, 16)] = zv
            return 0
        lax.fori_loop(0, 32, zh, 0)

    def row_body(i, _):
        r = wid * rpw + i
        pltpu.sync_copy(bm.at[r], bm_v)

        zero_hist()

        def bpass0(j, _):
            o = bm_v[pl.ds(j * 16, 16)]
            d = lax.convert_element_type(o >> jnp.uint32(24), jnp.int32)
            plsc.addupdate_scatter(hist, [d * 16 + lanes], ones)
            return 0
        with jax.named_scope("ph_blk0"):
            lax.fori_loop(0, NB // 16, bpass0, 0)
            b0, ab0, _ = hist_scan(jnp.int32(KTOP))

        zero_hist()
        b0u = lax.convert_element_type(b0, jnp.uint32)

        def bpass1(j, _):
            o = bm_v[pl.ds(j * 16, 16)]
            m = (o >> jnp.uint32(24)) == b0u
            d = lax.convert_element_type(
                (o >> jnp.uint32(16)) & jnp.uint32(0xFF), jnp.int32)
            plsc.addupdate_scatter(hist, [d * 16 + lanes], ones, mask=m)
            return 0
        with jax.named_scope("ph_blk1"):
            lax.fori_loop(0, NB // 16, bpass1, 0)
            kth1 = jnp.int32(KTOP) - ab0
            b1, _, _ = hist_scan(jnp.maximum(kth1, 1))
        bp = (b0 * 256 + b1)
        bpu = lax.convert_element_type(bp, jnp.uint32)

        def pre(j, _):
            cand_gid[pl.ds(j * 16, 16)] = jnp.full((16,), r * NB, jnp.int32)
            return 0
        lax.fori_loop(0, CAND_ROWS // 16, pre, 0)

        def bcomp(j, cnt):
            o = bm_v[pl.ds(j * 16, 16)]
            keep = (o >> jnp.uint32(16)) >= bpu
            ids = j * 16 + lanes
            plsc.store_compressed(cand_idx.at[pl.ds(cnt, 16)], ids, mask=keep)
            plsc.store_compressed(cand_gid.at[pl.ds(cnt, 16)],
                                  ids + r * NB, mask=keep)
            return cnt + popc(keep)
        with jax.named_scope("ph_bcomp"):
            cnt = lax.fori_loop(0, NB // 16, bcomp, jnp.int32(0))

        nchunks = (cnt + CH - 1) // CH

        def gat(c, _):
            pltpu.async_copy(
                keys2d.at[cand_gid.at[pl.ds(c * CH, CH)]],
                cand_buf.at[pl.ds(c * CH, CH)], sem)
            return 0
        with jax.named_scope("ph_gat"):
            lax.fori_loop(0, nchunks, gat, 0)

        def drain(c, _):
            pltpu.make_async_copy(
                keys2d.at[cand_gid.at[pl.ds(c * CH, CH)]],
                cand_buf.at[pl.ds(c * CH, CH)], sem).wait()
            return 0
        with jax.named_scope("ph_drain"):
            lax.fori_loop(0, nchunks, drain, 0)

        smem[0] = 0
        smem[1] = 0
        smem[2] = 24
        smem[3] = 0
        smem[4] = 0

        for lev in range(4):
            sh = 24 - 8 * lev

            @pl.when(smem[0] == 0)
            def _level():
                zero_hist()
                pu = lax.convert_element_type(smem[1], jnp.uint32)

                def epass(row, _):
                    for sub in range(8):
                        o = cand_buf[row, pl.ds(sub * 16, 16)]
                        d = lax.convert_element_type(
                            (o >> jnp.uint32(sh)) & jnp.uint32(0xFF),
                            jnp.int32)
                        if lev == 0:
                            plsc.addupdate_scatter(
                                hist, [d * 16 + lanes], ones)
                        else:
                            m = (o >> jnp.uint32(sh + 8)) == pu
                            plsc.addupdate_scatter(
                                hist, [d * 16 + lanes], ones, mask=m)
                    return 0
                with jax.named_scope("ph_epass%d" % lev):
                    lax.fori_loop(0, cnt, epass, 0)

                kneed = jnp.int32(KTOP) - smem[3]
                bb, above, inb = hist_scan(kneed)
                smem[1] = smem[1] * 256 + bb
                smem[2] = sh
                smem[3] = smem[3] + above
                fits = (smem[3] + inb) <= SEL

                @pl.when(fits)
                def _():
                    smem[0] = 1
                    smem[4] = 0
                if lev == 3:
                    @pl.when(jnp.logical_not(fits))
                    def _():
                        smem[0] = 1
                        smem[4] = 1

        def selpre(j, _):
            sel_o[pl.ds(j * 16, 16)] = jnp.zeros((16,), jnp.int32)
            sel_gi[pl.ds(j * 16, 16)] = (
                jnp.int32(0x7FFF0000) + j * 16 + lanes)
            return 0
        lax.fori_loop(0, SEL // 16, selpre, 0)

        pfin = lax.convert_element_type(smem[1], jnp.uint32)
        shfin = lax.convert_element_type(smem[2], jnp.uint32)

        @pl.when(smem[4] == 0)
        def _rank_compact():
            def cpass(row, ns):
                for sub in range(8):
                    o = cand_buf[row, pl.ds(sub * 16, 16)]
                    keep = (o >> shfin) >= pfin
                    pos = row * 128 + (sub * 16 + lanes)
                    plsc.store_compressed(
                        sel_o.at[pl.ds(ns, 16)],
                        plsc.bitcast(o, jnp.int32), mask=keep)
                    plsc.store_compressed(sel_gi.at[pl.ds(ns, 16)],
                                          pos, mask=keep)
                    ns = ns + popc(keep)
                return ns
            with jax.named_scope("ph_rcomp"):
                lax.fori_loop(0, cnt, cpass, jnp.int32(0))

        @pl.when(smem[4] == 1)
        def _equals_compact():
            need = jnp.int32(KTOP) - smem[3]

            def cpass(row, c):
                ns, ne = c
                for sub in range(8):
                    o = cand_buf[row, pl.ds(sub * 16, 16)]
                    pos = row * 128 + (sub * 16 + lanes)
                    keep = o > pfin
                    plsc.store_compressed(
                        sel_o.at[pl.ds(ns, 16)],
                        plsc.bitcast(o, jnp.int32), mask=keep)
                    plsc.store_compressed(sel_gi.at[pl.ds(ns, 16)],
                                          pos, mask=keep)
                    ns = ns + popc(keep)
                    eq = o == pfin
                    pref = plsc.cumsum(jnp.where(eq, 1, 0))
                    wm = jnp.logical_and(eq, (ne + pref) <= need)
                    plsc.store_compressed(
                        sel_o.at[pl.ds(ns, 16)],
                        plsc.bitcast(o, jnp.int32), mask=wm)
                    plsc.store_compressed(sel_gi.at[pl.ds(ns, 16)],
                                          pos, mask=wm)
                    npc = popc(wm)
                    ns = ns + npc
                    ne = ne + npc
                return (ns, ne)
            lax.fori_loop(0, cnt, cpass,
                          (smem[3] + jnp.int32(0), jnp.int32(0)))

        def rank_i(ii, _):
            iv = jnp.full((16,), ii, jnp.int32)
            oib = plsc.bitcast(plsc.load_gather(sel_o, [iv]), jnp.uint32)
            gib = plsc.load_gather(sel_gi, [iv])

            rank = jnp.int32(0)
            for j in range(SEL // 16):
                o = plsc.bitcast(sel_o[pl.ds(j * 16, 16)], jnp.uint32)
                g = sel_gi[pl.ds(j * 16, 16)]
                beat = jnp.logical_or(
                    o > oib, jnp.logical_and(o == oib, g < gib))
                rank = rank + popc(beat)
            u = jnp.where(oib >= jnp.uint32(0x80000000),
                          oib ^ jnp.uint32(0x80000000), jnp.uint32(0) - oib)
            val_v = plsc.bitcast(u, jnp.float32)
            rv = jnp.full((16,), rank, jnp.int32)
            lm = lanes == 0
            plsc.store_scatter(stage_v, [rv], val_v, mask=lm)
            plsc.store_scatter(stage_i, [rv], gib, mask=lm)
            return 0
        with jax.named_scope("ph_rank"):
            lax.fori_loop(0, SEL, rank_i, 0)

        for j in range(SEL // 16):
            pos = stage_i[pl.ds(j * 16, 16)]
            bslot = jnp.minimum(pos >> 7, jnp.int32(CAND_ROWS - 1))
            blk = plsc.load_gather(cand_idx, [bslot])
            stage_i[pl.ds(j * 16, 16)] = blk * 128 + (pos & 127)

        pltpu.sync_copy(stage_v, vals_ref.at[r])
        pltpu.sync_copy(stage_i, gidx_ref.at[r])
        return 0

    lax.fori_loop(0, rpw, row_body, 0)


def kernel(query_embeddings, item_embeddings_t, item_ids, k):
    bq, d = query_embeddings.shape
    nx = item_embeddings_t.shape[1]
    rt = 32

    items_p = jnp.pad(item_embeddings_t, ((0, 0), (0, XP - nx)))
    keys, bm = pl.pallas_call(
        functools.partial(_tc_body, nx),
        grid=(bq // rt,),
        in_specs=[
            pl.BlockSpec((rt, d), lambda i: (i, 0)),
            pl.BlockSpec((d, XP), lambda i: (0, 0)),
        ],
        out_specs=[
            pl.BlockSpec((rt, XP), lambda i: (i, 0)),
            pl.BlockSpec((rt, NB), lambda i: (i, 0)),
        ],
        out_shape=[
            jax.ShapeDtypeStruct((bq, XP), jnp.uint32),
            jax.ShapeDtypeStruct((bq, NB), jnp.uint32),
        ],
    )(query_embeddings, items_p)

    keys2d = keys.reshape(bq * NB, 128)
    mesh = plsc.VectorSubcoreMesh(core_axis_name="c", subcore_axis_name="s")
    vals128, gidx128 = pl.kernel(
        _sc_topk,
        mesh=mesh,
        compiler_params=pltpu.CompilerParams(needs_layout_passes=False),
        out_type=[
            jax.ShapeDtypeStruct((bq, SEL), jnp.float32),
            jax.ShapeDtypeStruct((bq, SEL), jnp.int32),
        ],
        scratch_types=[
            pltpu.VMEM((CAND_ROWS, 128), jnp.uint32),
            pltpu.VMEM((NB,), jnp.uint32),
            pltpu.VMEM((CAND_ROWS,), jnp.int32),
            pltpu.VMEM((CAND_ROWS,), jnp.int32),
            pltpu.VMEM((4096,), jnp.int32),
            pltpu.VMEM((SEL,), jnp.int32),
            pltpu.VMEM((SEL,), jnp.int32),
            pltpu.VMEM((SEL,), jnp.float32),
            pltpu.VMEM((SEL,), jnp.int32),
            pltpu.SMEM((8,), jnp.int32),
            pltpu.SemaphoreType.DMA,
        ],
    )(keys2d, bm)

    topk_logits = vals128[:, :KTOP]
    topk_indices = gidx128[:, :KTOP]
    topk_logits = jnp.nan_to_num(topk_logits, nan=-1000000000.0,
                                 posinf=1000000000.0, neginf=-1000000000.0)
    topk_indices = topk_indices + jnp.asarray(k - KTOP, dtype=jnp.int32)
    topk_indices = jnp.clip(topk_indices, 0, nx - 1)
    topk_item_ids = jnp.take(item_ids[0], topk_indices, axis=0)
    return (topk_logits, topk_item_ids)


# speedup vs baseline: 1.5499x; 1.5499x over previous
import functools

import jax
import jax.numpy as jnp
from jax import lax
from jax.experimental import pallas as pl
from jax.experimental.pallas import tpu as pltpu
from jax.experimental.pallas import tpu_sc as plsc

XP = 100352
NB = XP // 128
KTOP = 100
NWORK = 32
CH = 64
CAND_ROWS = 832
SEL = 128


def _order_u32(x):
    u = lax.bitcast_convert_type(x, jnp.uint32)
    neg = u >= jnp.uint32(0x80000000)
    return jnp.where(neg, jnp.uint32(0) - u, u ^ jnp.uint32(0x80000000))


def _tc_body(nx, q_ref, it_ref, keys_ref, bm_ref):
    logits = jnp.dot(q_ref[...], it_ref[...],
                     preferred_element_type=jnp.float32)
    col = lax.broadcasted_iota(jnp.int32, logits.shape, 1)
    logits = jnp.where(col < nx, logits, jnp.float32(-1e30))
    keys_ref[...] = _order_u32(logits)
    r = logits.reshape(logits.shape[0], NB, 128)
    bm_ref[...] = _order_u32(jnp.max(r, axis=2))


def _sc_topk(keys2d, bm, vals_ref, gidx_ref,
             cand_buf, bm_v, cand_idx, cand_gid, hist, sel_o, sel_gi,
             stage_v, stage_i, smem, sem):
    nc = 2
    wid = lax.axis_index("s") * nc + lax.axis_index("c")
    lanes = lax.iota(jnp.int32, 16)
    lanes256 = lanes * 256
    ones = jnp.ones((16,), jnp.int32)
    rpw = 1024 // NWORK

    def popc(mask):
        return plsc.all_reduce_population_count(mask)[0]

    def group_tot(g):
        t = hist[pl.ds(g * 16, 16)]
        for l in range(1, 16):
            t = t + hist[pl.ds(l * 256 + g * 16, 16)]
        return t

    def hist_scan(kth):
        def cond(c):
            g, cum, stop = c
            return jnp.logical_and(stop == 0, g >= 0)

        def body(c):
            g, cum, stop = c
            gsum = jnp.sum(group_tot(g))
            hit = (cum + gsum) >= kth
            return (jnp.where(hit, g, g - 1),
                    jnp.where(hit, cum, cum + gsum),
                    jnp.where(hit, 1, 0).astype(jnp.int32))

        g, cum, _ = lax.while_loop(cond, body, (jnp.int32(15),
                                                jnp.int32(0),
                                                jnp.int32(0)))
        g = jnp.maximum(g, 0)
        totv = group_tot(g)
        rsfx = plsc.cumsum(lax.rev(totv, (0,)))
        revmask = (cum + rsfx) >= kth
        f = plsc.all_reduce_ffs(revmask)
        f0 = f[0] if getattr(f, 'ndim', 0) else f
        i_star = 15 - f0
        onehot_r = lanes == f0
        onehot = lanes == i_star
        inb = jnp.sum(jnp.where(onehot, totv, 0))
        sstar = jnp.sum(jnp.where(onehot_r, rsfx, 0))
        return g * 16 + i_star, cum + sstar - inb, inb

    def zero_hist():
        zv = jnp.zeros((16,), jnp.int32)

        def zh(i, _):
            for s in range(8):
                hist[pl.ds(i * 128 + s * 16, 16)] = zv
            return 0
        lax.fori_loop(0, 32, zh, 0)

    def row_body(i, _):
        r = wid * rpw + i
        pltpu.sync_copy(bm.at[r], bm_v)

        zero_hist()

        def bpass0(j, _):
            o = bm_v[pl.ds(j * 16, 16)]
            d = lax.convert_element_type(o >> jnp.uint32(24), jnp.int32)
            plsc.addupdate_scatter(hist, [lanes256 + d], ones)
            return 0
        with jax.named_scope("ph_blk0"):
            lax.fori_loop(0, NB // 16, bpass0, 0)
            b0, ab0, _ = hist_scan(jnp.int32(KTOP))

        zero_hist()
        b0u = lax.convert_element_type(b0, jnp.uint32)

        def bpass1(j, _):
            o = bm_v[pl.ds(j * 16, 16)]
            m = (o >> jnp.uint32(24)) == b0u
            d = lax.convert_element_type(
                (o >> jnp.uint32(16)) & jnp.uint32(0xFF), jnp.int32)
            plsc.addupdate_scatter(hist, [lanes256 + d], ones, mask=m)
            return 0
        with jax.named_scope("ph_blk1"):
            lax.fori_loop(0, NB // 16, bpass1, 0)
            kth1 = jnp.int32(KTOP) - ab0
            b1, _, _ = hist_scan(jnp.maximum(kth1, 1))
        bp = (b0 * 256 + b1)
        bpu = lax.convert_element_type(bp, jnp.uint32)

        def pre(j, _):
            cand_gid[pl.ds(j * 16, 16)] = jnp.full((16,), r * NB, jnp.int32)
            return 0
        lax.fori_loop(0, CAND_ROWS // 16, pre, 0)

        def bcomp(j, cnt):
            o = bm_v[pl.ds(j * 16, 16)]
            keep = (o >> jnp.uint32(16)) >= bpu
            ids = j * 16 + lanes
            plsc.store_compressed(cand_idx.at[pl.ds(cnt, 16)], ids, mask=keep)
            plsc.store_compressed(cand_gid.at[pl.ds(cnt, 16)],
                                  ids + r * NB, mask=keep)
            return cnt + popc(keep)
        with jax.named_scope("ph_bcomp"):
            cnt = plsc.parallel_loop(0, NB // 16, carry=jnp.int32(0))(bcomp)

        nchunks = (cnt + CH - 1) // CH

        def gat(c, _):
            pltpu.async_copy(
                keys2d.at[cand_gid.at[pl.ds(c * CH, CH)]],
                cand_buf.at[pl.ds(c * CH, CH)], sem)
            return 0
        with jax.named_scope("ph_gat"):
            lax.fori_loop(0, nchunks, gat, 0)

        def drain(c, _):
            pltpu.make_async_copy(
                keys2d.at[cand_gid.at[pl.ds(c * CH, CH)]],
                cand_buf.at[pl.ds(c * CH, CH)], sem).wait()
            return 0
        with jax.named_scope("ph_drain"):
            lax.fori_loop(0, nchunks, drain, 0)

        smem[0] = 0
        smem[1] = jnp.where(ab0 == 0, b0, 0)
        smem[2] = 24
        smem[3] = 0
        smem[4] = 0
        smem[6] = jnp.where(ab0 == 0, 1, 0)

        for lev in range(4):
            sh = 24 - 8 * lev
            if lev == 0:
                run_lev = jnp.logical_and(smem[0] == 0, smem[6] == 0)
            else:
                run_lev = smem[0] == 0

            @pl.when(run_lev)
            def _level():
                zero_hist()
                pu = lax.convert_element_type(smem[1], jnp.uint32)

                def epass(row, _):
                    for sub in range(8):
                        o = cand_buf[row, pl.ds(sub * 16, 16)]
                        d = lax.convert_element_type(
                            (o >> jnp.uint32(sh)) & jnp.uint32(0xFF),
                            jnp.int32)
                        if lev == 0:
                            plsc.addupdate_scatter(
                                hist, [lanes256 + d], ones)
                        else:
                            m = (o >> jnp.uint32(sh + 8)) == pu
                            plsc.addupdate_scatter(
                                hist, [lanes256 + d], ones, mask=m)
                    return 0
                with jax.named_scope("ph_epass%d" % lev):
                    plsc.parallel_loop(0, cnt)(lambda row: epass(row, 0) and None)

                kneed = jnp.int32(KTOP) - smem[3]
                bb, above, inb = hist_scan(kneed)
                smem[1] = smem[1] * 256 + bb
                smem[2] = sh
                smem[3] = smem[3] + above
                fits = (smem[3] + inb) <= SEL

                @pl.when(fits)
                def _():
                    smem[0] = 1
                    smem[4] = 0
                if lev == 3:
                    @pl.when(jnp.logical_not(fits))
                    def _():
                        smem[0] = 1
                        smem[4] = 1

        def selpre(j, _):
            sel_o[pl.ds(j * 16, 16)] = jnp.zeros((16,), jnp.int32)
            sel_gi[pl.ds(j * 16, 16)] = (
                jnp.int32(0x7FFF0000) + j * 16 + lanes)
            return 0
        lax.fori_loop(0, SEL // 16, selpre, 0)

        pfin = lax.convert_element_type(smem[1], jnp.uint32)
        shfin = lax.convert_element_type(smem[2], jnp.uint32)

        @pl.when(smem[4] == 0)
        def _rank_compact():
            def cpass(row, ns):
                for sub in range(8):
                    o = cand_buf[row, pl.ds(sub * 16, 16)]
                    keep = (o >> shfin) >= pfin
                    pos = row * 128 + (sub * 16 + lanes)
                    plsc.store_compressed(
                        sel_o.at[pl.ds(ns, 16)],
                        plsc.bitcast(o, jnp.int32), mask=keep)
                    plsc.store_compressed(sel_gi.at[pl.ds(ns, 16)],
                                          pos, mask=keep)
                    ns = ns + popc(keep)
                return ns
            with jax.named_scope("ph_rcomp"):
                plsc.parallel_loop(0, cnt, carry=jnp.int32(0))(cpass)

        @pl.when(smem[4] == 1)
        def _equals_compact():
            need = jnp.int32(KTOP) - smem[3]

            def cpass(row, c):
                ns, ne = c
                for sub in range(8):
                    o = cand_buf[row, pl.ds(sub * 16, 16)]
                    pos = row * 128 + (sub * 16 + lanes)
                    keep = o > pfin
                    plsc.store_compressed(
                        sel_o.at[pl.ds(ns, 16)],
                        plsc.bitcast(o, jnp.int32), mask=keep)
                    plsc.store_compressed(sel_gi.at[pl.ds(ns, 16)],
                                          pos, mask=keep)
                    ns = ns + popc(keep)
                    eq = o == pfin
                    pref = plsc.cumsum(jnp.where(eq, 1, 0))
                    wm = jnp.logical_and(eq, (ne + pref) <= need)
                    plsc.store_compressed(
                        sel_o.at[pl.ds(ns, 16)],
                        plsc.bitcast(o, jnp.int32), mask=wm)
                    plsc.store_compressed(sel_gi.at[pl.ds(ns, 16)],
                                          pos, mask=wm)
                    npc = popc(wm)
                    ns = ns + npc
                    ne = ne + npc
                return (ns, ne)
            lax.fori_loop(0, cnt, cpass,
                          (smem[3] + jnp.int32(0), jnp.int32(0)))

        def rank_i(ii, _):
            iv = jnp.full((16,), ii, jnp.int32)
            oib = plsc.bitcast(plsc.load_gather(sel_o, [iv]), jnp.uint32)
            gib = plsc.load_gather(sel_gi, [iv])

            rank = jnp.int32(0)
            for j in range(SEL // 16):
                o = plsc.bitcast(sel_o[pl.ds(j * 16, 16)], jnp.uint32)
                g = sel_gi[pl.ds(j * 16, 16)]
                beat = jnp.logical_or(
                    o > oib, jnp.logical_and(o == oib, g < gib))
                rank = rank + popc(beat)
            u = jnp.where(oib >= jnp.uint32(0x80000000),
                          oib ^ jnp.uint32(0x80000000), jnp.uint32(0) - oib)
            val_v = plsc.bitcast(u, jnp.float32)
            rv = jnp.full((16,), rank, jnp.int32)
            lm = lanes == 0
            plsc.store_scatter(stage_v, [rv], val_v, mask=lm)
            plsc.store_scatter(stage_i, [rv], gib, mask=lm)
            return 0
        with jax.named_scope("ph_rank"):
            plsc.parallel_loop(0, SEL)(lambda ii: rank_i(ii, 0) and None)

        for j in range(SEL // 16):
            pos = stage_i[pl.ds(j * 16, 16)]
            bslot = jnp.minimum(pos >> 7, jnp.int32(CAND_ROWS - 1))
            blk = plsc.load_gather(cand_idx, [bslot])
            stage_i[pl.ds(j * 16, 16)] = blk * 128 + (pos & 127)

        pltpu.sync_copy(stage_v, vals_ref.at[r])
        pltpu.sync_copy(stage_i, gidx_ref.at[r])
        return 0

    lax.fori_loop(0, rpw, row_body, 0)


def kernel(query_embeddings, item_embeddings_t, item_ids, k):
    bq, d = query_embeddings.shape
    nx = item_embeddings_t.shape[1]
    rt = 32

    items_p = jnp.pad(item_embeddings_t, ((0, 0), (0, XP - nx)))
    keys, bm = pl.pallas_call(
        functools.partial(_tc_body, nx),
        grid=(bq // rt,),
        in_specs=[
            pl.BlockSpec((rt, d), lambda i: (i, 0)),
            pl.BlockSpec((d, XP), lambda i: (0, 0)),
        ],
        out_specs=[
            pl.BlockSpec((rt, XP), lambda i: (i, 0)),
            pl.BlockSpec((rt, NB), lambda i: (i, 0)),
        ],
        out_shape=[
            jax.ShapeDtypeStruct((bq, XP), jnp.uint32),
            jax.ShapeDtypeStruct((bq, NB), jnp.uint32),
        ],
    )(query_embeddings, items_p)

    keys2d = keys.reshape(bq * NB, 128)
    mesh = plsc.VectorSubcoreMesh(core_axis_name="c", subcore_axis_name="s")
    vals128, gidx128 = pl.kernel(
        _sc_topk,
        mesh=mesh,
        compiler_params=pltpu.CompilerParams(needs_layout_passes=False),
        out_type=[
            jax.ShapeDtypeStruct((bq, SEL), jnp.float32),
            jax.ShapeDtypeStruct((bq, SEL), jnp.int32),
        ],
        scratch_types=[
            pltpu.VMEM((CAND_ROWS, 128), jnp.uint32),
            pltpu.VMEM((NB,), jnp.uint32),
            pltpu.VMEM((CAND_ROWS,), jnp.int32),
            pltpu.VMEM((CAND_ROWS,), jnp.int32),
            pltpu.VMEM((4096,), jnp.int32),
            pltpu.VMEM((SEL,), jnp.int32),
            pltpu.VMEM((SEL,), jnp.int32),
            pltpu.VMEM((SEL,), jnp.float32),
            pltpu.VMEM((SEL,), jnp.int32),
            pltpu.SMEM((8,), jnp.int32),
            pltpu.SemaphoreType.DMA,
        ],
    )(keys2d, bm)

    topk_logits = vals128[:, :KTOP]
    topk_indices = gidx128[:, :KTOP]
    topk_logits = jnp.nan_to_num(topk_logits, nan=-1000000000.0,
                                 posinf=1000000000.0, neginf=-1000000000.0)
    topk_indices = topk_indices + jnp.asarray(k - KTOP, dtype=jnp.int32)
    topk_indices = jnp.clip(topk_indices, 0, nx - 1)
    topk_item_ids = jnp.take(item_ids[0], topk_indices, axis=0)
    return (topk_logits, topk_item_ids)


# speedup vs baseline: 1.8615x; 1.2010x over previous
import functools

import jax
import jax.numpy as jnp
from jax import lax
from jax.experimental import pallas as pl
from jax.experimental.pallas import tpu as pltpu
from jax.experimental.pallas import tpu_sc as plsc

XP = 100352
NB = XP // 128
KTOP = 100
NWORK = 32
CH = 64
CAND_ROWS = 832
SEL = 128


def _order_u32(x):
    u = lax.bitcast_convert_type(x, jnp.uint32)
    neg = u >= jnp.uint32(0x80000000)
    return jnp.where(neg, jnp.uint32(0) - u, u ^ jnp.uint32(0x80000000))


def _tc_body(nx, q_ref, it_ref, keys_ref, bm_ref):
    logits = jnp.dot(q_ref[...], it_ref[...],
                     preferred_element_type=jnp.float32)
    col = lax.broadcasted_iota(jnp.int32, logits.shape, 1)
    logits = jnp.where(col < nx, logits, jnp.float32(-1e30))
    keys_ref[...] = _order_u32(logits)
    r = logits.reshape(logits.shape[0], NB, 128)
    bm_ref[...] = _order_u32(jnp.max(r, axis=2))


def _sc_topk(rpw, keys2d, bm, vals_ref, gidx_ref,
             cand_buf, bm_v, cand_idx, cand_gid, hist, sel_o, sel_gi,
             stage_v, stage_i, smem, sem):
    nc = 2
    wid = lax.axis_index("s") * nc + lax.axis_index("c")
    lanes = lax.iota(jnp.int32, 16)
    lanes256 = lanes * 256
    ones = jnp.ones((16,), jnp.int32)

    def popc(mask):
        return plsc.all_reduce_population_count(mask)[0]

    def group_tot(g):
        t = hist[pl.ds(g * 16, 16)]
        for l in range(1, 16):
            t = t + hist[pl.ds(l * 256 + g * 16, 16)]
        return t

    def hist_scan(kth):
        def cond(c):
            g, cum, stop = c
            return jnp.logical_and(stop == 0, g >= 0)

        def body(c):
            g, cum, stop = c
            gsum = jnp.sum(group_tot(g))
            hit = (cum + gsum) >= kth
            return (jnp.where(hit, g, g - 1),
                    jnp.where(hit, cum, cum + gsum),
                    jnp.where(hit, 1, 0).astype(jnp.int32))

        g, cum, _ = lax.while_loop(cond, body, (jnp.int32(15),
                                                jnp.int32(0),
                                                jnp.int32(0)))
        g = jnp.maximum(g, 0)
        totv = group_tot(g)
        rsfx = plsc.cumsum(lax.rev(totv, (0,)))
        revmask = (cum + rsfx) >= kth
        f = plsc.all_reduce_ffs(revmask)
        f0 = f[0] if getattr(f, 'ndim', 0) else f
        i_star = 15 - f0
        onehot_r = lanes == f0
        onehot = lanes == i_star
        inb = jnp.sum(jnp.where(onehot, totv, 0))
        sstar = jnp.sum(jnp.where(onehot_r, rsfx, 0))
        return g * 16 + i_star, cum + sstar - inb, inb

    def zero_hist():
        zv = jnp.zeros((16,), jnp.int32)

        def zh(i, _):
            for s in range(8):
                hist[pl.ds(i * 128 + s * 16, 16)] = zv
            return 0
        lax.fori_loop(0, 32, zh, 0)

    def row_body(i, _):
        r = wid * rpw + i
        pltpu.sync_copy(bm.at[r], bm_v)

        zero_hist()

        def bpass0(j, _):
            o = bm_v[pl.ds(j * 16, 16)]
            d = lax.convert_element_type(o >> jnp.uint32(24), jnp.int32)
            plsc.addupdate_scatter(hist, [lanes256 + d], ones)
            return 0
        with jax.named_scope("ph_blk0"):
            lax.fori_loop(0, NB // 16, bpass0, 0)
            b0, ab0, _ = hist_scan(jnp.int32(KTOP))

        zero_hist()
        b0u = lax.convert_element_type(b0, jnp.uint32)

        def bpass1(j, _):
            o = bm_v[pl.ds(j * 16, 16)]
            m = (o >> jnp.uint32(24)) == b0u
            d = lax.convert_element_type(
                (o >> jnp.uint32(16)) & jnp.uint32(0xFF), jnp.int32)
            plsc.addupdate_scatter(hist, [lanes256 + d], ones, mask=m)
            return 0
        with jax.named_scope("ph_blk1"):
            lax.fori_loop(0, NB // 16, bpass1, 0)
            kth1 = jnp.int32(KTOP) - ab0
            b1, _, _ = hist_scan(jnp.maximum(kth1, 1))
        bp = (b0 * 256 + b1)
        bpu = lax.convert_element_type(bp, jnp.uint32)

        def pre(j, _):
            cand_gid[pl.ds(j * 16, 16)] = jnp.full((16,), r * NB, jnp.int32)
            return 0
        lax.fori_loop(0, CAND_ROWS // 16, pre, 0)

        def bcomp(j, cnt):
            o = bm_v[pl.ds(j * 16, 16)]
            keep = (o >> jnp.uint32(16)) >= bpu
            ids = j * 16 + lanes
            plsc.store_compressed(cand_idx.at[pl.ds(cnt, 16)], ids, mask=keep)
            plsc.store_compressed(cand_gid.at[pl.ds(cnt, 16)],
                                  ids + r * NB, mask=keep)
            return cnt + popc(keep)
        with jax.named_scope("ph_bcomp"):
            cnt = plsc.parallel_loop(0, NB // 16, carry=jnp.int32(0))(bcomp)

        nchunks = (cnt + CH - 1) // CH

        def gat(c, _):
            pltpu.async_copy(
                keys2d.at[cand_gid.at[pl.ds(c * CH, CH)]],
                cand_buf.at[pl.ds(c * CH, CH)], sem)
            return 0
        with jax.named_scope("ph_gat"):
            lax.fori_loop(0, nchunks, gat, 0)

        def drain(c, _):
            pltpu.make_async_copy(
                keys2d.at[cand_gid.at[pl.ds(c * CH, CH)]],
                cand_buf.at[pl.ds(c * CH, CH)], sem).wait()
            return 0
        with jax.named_scope("ph_drain"):
            lax.fori_loop(0, nchunks, drain, 0)

        smem[0] = 0
        smem[1] = jnp.where(ab0 == 0, b0, 0)
        smem[2] = 24
        smem[3] = 0
        smem[4] = 0
        smem[6] = jnp.where(ab0 == 0, 1, 0)

        for lev in range(4):
            sh = 24 - 8 * lev
            if lev == 0:
                run_lev = jnp.logical_and(smem[0] == 0, smem[6] == 0)
            else:
                run_lev = smem[0] == 0

            @pl.when(run_lev)
            def _level():
                zero_hist()
                pu = lax.convert_element_type(smem[1], jnp.uint32)

                def epass(row, _):
                    for sub in range(8):
                        o = cand_buf[row, pl.ds(sub * 16, 16)]
                        d = lax.convert_element_type(
                            (o >> jnp.uint32(sh)) & jnp.uint32(0xFF),
                            jnp.int32)
                        if lev == 0:
                            plsc.addupdate_scatter(
                                hist, [lanes256 + d], ones)
                        else:
                            m = (o >> jnp.uint32(sh + 8)) == pu
                            plsc.addupdate_scatter(
                                hist, [lanes256 + d], ones, mask=m)
                    return 0
                with jax.named_scope("ph_epass%d" % lev):
                    plsc.parallel_loop(0, cnt)(lambda row: epass(row, 0) and None)

                kneed = jnp.int32(KTOP) - smem[3]
                bb, above, inb = hist_scan(kneed)
                smem[1] = smem[1] * 256 + bb
                smem[2] = sh
                smem[3] = smem[3] + above
                fits = (smem[3] + inb) <= SEL

                @pl.when(fits)
                def _():
                    smem[0] = 1
                    smem[4] = 0
                if lev == 3:
                    @pl.when(jnp.logical_not(fits))
                    def _():
                        smem[0] = 1
                        smem[4] = 1

        def selpre(j, _):
            sel_o[pl.ds(j * 16, 16)] = jnp.zeros((16,), jnp.int32)
            sel_gi[pl.ds(j * 16, 16)] = (
                jnp.int32(0x7FFF0000) + j * 16 + lanes)
            return 0
        lax.fori_loop(0, SEL // 16, selpre, 0)

        pfin = lax.convert_element_type(smem[1], jnp.uint32)
        shfin = lax.convert_element_type(smem[2], jnp.uint32)

        @pl.when(smem[4] == 0)
        def _rank_compact():
            def cpass(row, ns):
                for sub in range(8):
                    o = cand_buf[row, pl.ds(sub * 16, 16)]
                    keep = (o >> shfin) >= pfin
                    pos = row * 128 + (sub * 16 + lanes)
                    plsc.store_compressed(
                        sel_o.at[pl.ds(ns, 16)],
                        plsc.bitcast(o, jnp.int32), mask=keep)
                    plsc.store_compressed(sel_gi.at[pl.ds(ns, 16)],
                                          pos, mask=keep)
                    ns = ns + popc(keep)
                return ns
            with jax.named_scope("ph_rcomp"):
                plsc.parallel_loop(0, cnt, carry=jnp.int32(0))(cpass)

        @pl.when(smem[4] == 1)
        def _equals_compact():
            need = jnp.int32(KTOP) - smem[3]

            def cpass(row, c):
                ns, ne = c
                for sub in range(8):
                    o = cand_buf[row, pl.ds(sub * 16, 16)]
                    pos = row * 128 + (sub * 16 + lanes)
                    keep = o > pfin
                    plsc.store_compressed(
                        sel_o.at[pl.ds(ns, 16)],
                        plsc.bitcast(o, jnp.int32), mask=keep)
                    plsc.store_compressed(sel_gi.at[pl.ds(ns, 16)],
                                          pos, mask=keep)
                    ns = ns + popc(keep)
                    eq = o == pfin
                    pref = plsc.cumsum(jnp.where(eq, 1, 0))
                    wm = jnp.logical_and(eq, (ne + pref) <= need)
                    plsc.store_compressed(
                        sel_o.at[pl.ds(ns, 16)],
                        plsc.bitcast(o, jnp.int32), mask=wm)
                    plsc.store_compressed(sel_gi.at[pl.ds(ns, 16)],
                                          pos, mask=wm)
                    npc = popc(wm)
                    ns = ns + npc
                    ne = ne + npc
                return (ns, ne)
            lax.fori_loop(0, cnt, cpass,
                          (smem[3] + jnp.int32(0), jnp.int32(0)))

        def rank_i(ii, _):
            iv = jnp.full((16,), ii, jnp.int32)
            oib = plsc.bitcast(plsc.load_gather(sel_o, [iv]), jnp.uint32)
            gib = plsc.load_gather(sel_gi, [iv])

            rank = jnp.int32(0)
            for j in range(SEL // 16):
                o = plsc.bitcast(sel_o[pl.ds(j * 16, 16)], jnp.uint32)
                g = sel_gi[pl.ds(j * 16, 16)]
                beat = jnp.logical_or(
                    o > oib, jnp.logical_and(o == oib, g < gib))
                rank = rank + popc(beat)
            u = jnp.where(oib >= jnp.uint32(0x80000000),
                          oib ^ jnp.uint32(0x80000000), jnp.uint32(0) - oib)
            val_v = plsc.bitcast(u, jnp.float32)
            rv = jnp.full((16,), rank, jnp.int32)
            lm = lanes == 0
            plsc.store_scatter(stage_v, [rv], val_v, mask=lm)
            plsc.store_scatter(stage_i, [rv], gib, mask=lm)
            return 0
        with jax.named_scope("ph_rank"):
            plsc.parallel_loop(0, SEL)(lambda ii: rank_i(ii, 0) and None)

        for j in range(SEL // 16):
            pos = stage_i[pl.ds(j * 16, 16)]
            bslot = jnp.minimum(pos >> 7, jnp.int32(CAND_ROWS - 1))
            blk = plsc.load_gather(cand_idx, [bslot])
            stage_i[pl.ds(j * 16, 16)] = blk * 128 + (pos & 127)

        pltpu.sync_copy(stage_v, vals_ref.at[r])
        pltpu.sync_copy(stage_i, gidx_ref.at[r])
        return 0

    lax.fori_loop(0, rpw, row_body, 0)


def kernel(query_embeddings, item_embeddings_t, item_ids, k):
    bq, d = query_embeddings.shape
    nx = item_embeddings_t.shape[1]
    rt = 32

    items_p = jnp.pad(item_embeddings_t, ((0, 0), (0, XP - nx)))
    mesh = plsc.VectorSubcoreMesh(core_axis_name="c", subcore_axis_name="s")
    ngrp = 2
    bg = bq // ngrp
    vparts, gparts = [], []
    for g in range(ngrp):
        qg = lax.slice_in_dim(query_embeddings, g * bg, (g + 1) * bg, axis=0)
        keys, bmax = pl.pallas_call(
            functools.partial(_tc_body, nx),
            grid=(bg // rt,),
            in_specs=[
                pl.BlockSpec((rt, d), lambda i: (i, 0)),
                pl.BlockSpec((d, XP), lambda i: (0, 0)),
            ],
            out_specs=[
                pl.BlockSpec((rt, XP), lambda i: (i, 0)),
                pl.BlockSpec((rt, NB), lambda i: (i, 0)),
            ],
            out_shape=[
                jax.ShapeDtypeStruct((bg, XP), jnp.uint32),
                jax.ShapeDtypeStruct((bg, NB), jnp.uint32),
            ],
        )(qg, items_p)
        keys2d = keys.reshape(bg * NB, 128)
        vg, gg = pl.kernel(
            functools.partial(_sc_topk, bg // NWORK),
            mesh=mesh,
            compiler_params=pltpu.CompilerParams(needs_layout_passes=False),
            out_type=[
                jax.ShapeDtypeStruct((bg, SEL), jnp.float32),
                jax.ShapeDtypeStruct((bg, SEL), jnp.int32),
            ],
            scratch_types=[
                pltpu.VMEM((CAND_ROWS, 128), jnp.uint32),
                pltpu.VMEM((NB,), jnp.uint32),
                pltpu.VMEM((CAND_ROWS,), jnp.int32),
                pltpu.VMEM((CAND_ROWS,), jnp.int32),
                pltpu.VMEM((4096,), jnp.int32),
                pltpu.VMEM((SEL,), jnp.int32),
                pltpu.VMEM((SEL,), jnp.int32),
                pltpu.VMEM((SEL,), jnp.float32),
                pltpu.VMEM((SEL,), jnp.int32),
                pltpu.SMEM((8,), jnp.int32),
                pltpu.SemaphoreType.DMA,
            ],
        )(keys2d, bmax)
        vparts.append(vg)
        gparts.append(gg)
    vals128 = jnp.concatenate(vparts, axis=0)
    gidx128 = jnp.concatenate(gparts, axis=0)

    topk_logits = vals128[:, :KTOP]
    topk_indices = gidx128[:, :KTOP]
    topk_logits = jnp.nan_to_num(topk_logits, nan=-1000000000.0,
                                 posinf=1000000000.0, neginf=-1000000000.0)
    topk_indices = topk_indices + jnp.asarray(k - KTOP, dtype=jnp.int32)
    topk_indices = jnp.clip(topk_indices, 0, nx - 1)
    topk_item_ids = jnp.take(item_ids[0], topk_indices, axis=0)
    return (topk_logits, topk_item_ids)


# speedup vs baseline: 2.0209x; 1.0856x over previous
import functools

import jax
import jax.numpy as jnp
from jax import lax
from jax.experimental import pallas as pl
from jax.experimental.pallas import tpu as pltpu
from jax.experimental.pallas import tpu_sc as plsc

XP = 100352
NB = XP // 128
KTOP = 100
NWORK = 32
CH = 64
CAND_ROWS = 832
SEL = 128


def _order_u32(x):
    u = lax.bitcast_convert_type(x, jnp.uint32)
    neg = u >= jnp.uint32(0x80000000)
    return jnp.where(neg, jnp.uint32(0) - u, u ^ jnp.uint32(0x80000000))


def _tc_body(nx, q_ref, it_ref, keys_ref, bm_ref):
    logits = jnp.dot(q_ref[...], it_ref[...],
                     preferred_element_type=jnp.float32)
    col = lax.broadcasted_iota(jnp.int32, logits.shape, 1)
    logits = jnp.where(col < nx, logits, jnp.float32(-1e30))
    keys_ref[...] = _order_u32(logits)
    r = logits.reshape(logits.shape[0], NB, 128)
    bm_ref[...] = _order_u32(jnp.max(r, axis=2))


def _sc_topk(rpw, keys2d, bm, vals_ref, gidx_ref,
             cand_buf, bm_v, cand_idx, cand_gid, hist, sel_o, sel_gi,
             stage_v, stage_i, smem, sem):
    nc = 2
    wid = lax.axis_index("s") * nc + lax.axis_index("c")
    lanes = lax.iota(jnp.int32, 16)
    lanes256 = lanes * 256
    ones = jnp.ones((16,), jnp.int32)

    def popc(mask):
        return plsc.all_reduce_population_count(mask)[0]

    def group_tot(g):
        t = hist[pl.ds(g * 16, 16)]
        for l in range(1, 16):
            t = t + hist[pl.ds(l * 256 + g * 16, 16)]
        return t

    def hist_scan(kth):
        def cond(c):
            g, cum, stop = c
            return jnp.logical_and(stop == 0, g >= 0)

        def body(c):
            g, cum, stop = c
            gsum = jnp.sum(group_tot(g))
            hit = (cum + gsum) >= kth
            return (jnp.where(hit, g, g - 1),
                    jnp.where(hit, cum, cum + gsum),
                    jnp.where(hit, 1, 0).astype(jnp.int32))

        g, cum, _ = lax.while_loop(cond, body, (jnp.int32(15),
                                                jnp.int32(0),
                                                jnp.int32(0)))
        g = jnp.maximum(g, 0)
        totv = group_tot(g)
        rsfx = plsc.cumsum(lax.rev(totv, (0,)))
        revmask = (cum + rsfx) >= kth
        f = plsc.all_reduce_ffs(revmask)
        f0 = f[0] if getattr(f, 'ndim', 0) else f
        i_star = 15 - f0
        onehot_r = lanes == f0
        onehot = lanes == i_star
        inb = jnp.sum(jnp.where(onehot, totv, 0))
        sstar = jnp.sum(jnp.where(onehot_r, rsfx, 0))
        return g * 16 + i_star, cum + sstar - inb, inb

    def zero_hist():
        zv = jnp.zeros((16,), jnp.int32)

        def zh(i, _):
            for s in range(8):
                hist[pl.ds(i * 128 + s * 16, 16)] = zv
            return 0
        lax.fori_loop(0, 32, zh, 0)

    def row_body(i, _):
        r = wid * rpw + i
        pltpu.sync_copy(bm.at[r], bm_v)

        zero_hist()

        def bpass0(j, _):
            o = bm_v[pl.ds(j * 16, 16)]
            d = lax.convert_element_type(o >> jnp.uint32(24), jnp.int32)
            plsc.addupdate_scatter(hist, [lanes256 + d], ones)
            return 0
        with jax.named_scope("ph_blk0"):
            lax.fori_loop(0, NB // 16, bpass0, 0)
            b0, ab0, _ = hist_scan(jnp.int32(KTOP))

        zero_hist()
        b0u = lax.convert_element_type(b0, jnp.uint32)

        def bpass1(j, _):
            o = bm_v[pl.ds(j * 16, 16)]
            m = (o >> jnp.uint32(24)) == b0u
            d = lax.convert_element_type(
                (o >> jnp.uint32(16)) & jnp.uint32(0xFF), jnp.int32)
            plsc.addupdate_scatter(hist, [lanes256 + d], ones, mask=m)
            return 0
        with jax.named_scope("ph_blk1"):
            lax.fori_loop(0, NB // 16, bpass1, 0)
            kth1 = jnp.int32(KTOP) - ab0
            b1, _, _ = hist_scan(jnp.maximum(kth1, 1))
        bp = (b0 * 256 + b1)
        bpu = lax.convert_element_type(bp, jnp.uint32)

        def pre(j, _):
            cand_gid[pl.ds(j * 16, 16)] = jnp.full((16,), r * NB, jnp.int32)
            return 0
        lax.fori_loop(0, CAND_ROWS // 16, pre, 0)

        def bcomp(j, cnt):
            o = bm_v[pl.ds(j * 16, 16)]
            keep = (o >> jnp.uint32(16)) >= bpu
            ids = j * 16 + lanes
            plsc.store_compressed(cand_idx.at[pl.ds(cnt, 16)], ids, mask=keep)
            plsc.store_compressed(cand_gid.at[pl.ds(cnt, 16)],
                                  ids + r * NB, mask=keep)
            return cnt + popc(keep)
        with jax.named_scope("ph_bcomp"):
            cnt = plsc.parallel_loop(0, NB // 16, carry=jnp.int32(0))(bcomp)

        nchunks = (cnt + CH - 1) // CH

        def gat(c, _):
            pltpu.async_copy(
                keys2d.at[cand_gid.at[pl.ds(c * CH, CH)]],
                cand_buf.at[pl.ds(c * CH, CH)], sem)
            return 0
        with jax.named_scope("ph_gat"):
            lax.fori_loop(0, nchunks, gat, 0)

        def drain(c, _):
            pltpu.make_async_copy(
                keys2d.at[cand_gid.at[pl.ds(c * CH, CH)]],
                cand_buf.at[pl.ds(c * CH, CH)], sem).wait()
            return 0
        with jax.named_scope("ph_drain"):
            lax.fori_loop(0, nchunks, drain, 0)

        smem[0] = 0
        smem[1] = jnp.where(ab0 == 0, b0, 0)
        smem[2] = 24
        smem[3] = 0
        smem[4] = 0
        smem[6] = jnp.where(ab0 == 0, 1, 0)

        for lev in range(4):
            sh = 24 - 8 * lev
            if lev == 0:
                run_lev = jnp.logical_and(smem[0] == 0, smem[6] == 0)
            else:
                run_lev = smem[0] == 0

            @pl.when(run_lev)
            def _level():
                zero_hist()
                pu = lax.convert_element_type(smem[1], jnp.uint32)

                def epass(row, _):
                    for sub in range(8):
                        o = cand_buf[row, pl.ds(sub * 16, 16)]
                        d = lax.convert_element_type(
                            (o >> jnp.uint32(sh)) & jnp.uint32(0xFF),
                            jnp.int32)
                        if lev == 0:
                            plsc.addupdate_scatter(
                                hist, [lanes256 + d], ones)
                        else:
                            m = (o >> jnp.uint32(sh + 8)) == pu
                            plsc.addupdate_scatter(
                                hist, [lanes256 + d], ones, mask=m)
                    return 0
                with jax.named_scope("ph_epass%d" % lev):
                    plsc.parallel_loop(0, cnt)(lambda row: epass(row, 0) and None)

                kneed = jnp.int32(KTOP) - smem[3]
                bb, above, inb = hist_scan(kneed)
                smem[1] = smem[1] * 256 + bb
                smem[2] = sh
                smem[3] = smem[3] + above
                fits = (smem[3] + inb) <= SEL

                @pl.when(fits)
                def _():
                    smem[0] = 1
                    smem[4] = 0
                if lev == 3:
                    @pl.when(jnp.logical_not(fits))
                    def _():
                        smem[0] = 1
                        smem[4] = 1

        def selpre(j, _):
            sel_o[pl.ds(j * 16, 16)] = jnp.zeros((16,), jnp.int32)
            sel_gi[pl.ds(j * 16, 16)] = (
                jnp.int32(0x7FFF0000) + j * 16 + lanes)
            return 0
        lax.fori_loop(0, SEL // 16, selpre, 0)

        pfin = lax.convert_element_type(smem[1], jnp.uint32)
        shfin = lax.convert_element_type(smem[2], jnp.uint32)

        @pl.when(smem[4] == 0)
        def _rank_compact():
            def cpass(row, ns):
                for sub in range(8):
                    o = cand_buf[row, pl.ds(sub * 16, 16)]
                    keep = (o >> shfin) >= pfin
                    pos = row * 128 + (sub * 16 + lanes)
                    plsc.store_compressed(
                        sel_o.at[pl.ds(ns, 16)],
                        plsc.bitcast(o, jnp.int32), mask=keep)
                    plsc.store_compressed(sel_gi.at[pl.ds(ns, 16)],
                                          pos, mask=keep)
                    ns = ns + popc(keep)
                return ns
            with jax.named_scope("ph_rcomp"):
                plsc.parallel_loop(0, cnt, carry=jnp.int32(0))(cpass)

        @pl.when(smem[4] == 1)
        def _equals_compact():
            need = jnp.int32(KTOP) - smem[3]

            def cpass(row, c):
                ns, ne = c
                for sub in range(8):
                    o = cand_buf[row, pl.ds(sub * 16, 16)]
                    pos = row * 128 + (sub * 16 + lanes)
                    keep = o > pfin
                    plsc.store_compressed(
                        sel_o.at[pl.ds(ns, 16)],
                        plsc.bitcast(o, jnp.int32), mask=keep)
                    plsc.store_compressed(sel_gi.at[pl.ds(ns, 16)],
                                          pos, mask=keep)
                    ns = ns + popc(keep)
                    eq = o == pfin
                    pref = plsc.cumsum(jnp.where(eq, 1, 0))
                    wm = jnp.logical_and(eq, (ne + pref) <= need)
                    plsc.store_compressed(
                        sel_o.at[pl.ds(ns, 16)],
                        plsc.bitcast(o, jnp.int32), mask=wm)
                    plsc.store_compressed(sel_gi.at[pl.ds(ns, 16)],
                                          pos, mask=wm)
                    npc = popc(wm)
                    ns = ns + npc
                    ne = ne + npc
                return (ns, ne)
            lax.fori_loop(0, cnt, cpass,
                          (smem[3] + jnp.int32(0), jnp.int32(0)))

        def rank_i(ii, _):
            iv = jnp.full((16,), ii, jnp.int32)
            oib = plsc.bitcast(plsc.load_gather(sel_o, [iv]), jnp.uint32)
            gib = plsc.load_gather(sel_gi, [iv])

            rank = jnp.int32(0)
            for j in range(SEL // 16):
                o = plsc.bitcast(sel_o[pl.ds(j * 16, 16)], jnp.uint32)
                g = sel_gi[pl.ds(j * 16, 16)]
                beat = jnp.logical_or(
                    o > oib, jnp.logical_and(o == oib, g < gib))
                rank = rank + popc(beat)
            u = jnp.where(oib >= jnp.uint32(0x80000000),
                          oib ^ jnp.uint32(0x80000000), jnp.uint32(0) - oib)
            val_v = plsc.bitcast(u, jnp.float32)
            rv = jnp.full((16,), rank, jnp.int32)
            lm = lanes == 0
            plsc.store_scatter(stage_v, [rv], val_v, mask=lm)
            plsc.store_scatter(stage_i, [rv], gib, mask=lm)
            return 0
        with jax.named_scope("ph_rank"):
            plsc.parallel_loop(0, SEL)(lambda ii: rank_i(ii, 0) and None)

        for j in range(SEL // 16):
            pos = stage_i[pl.ds(j * 16, 16)]
            bslot = jnp.minimum(pos >> 7, jnp.int32(CAND_ROWS - 1))
            blk = plsc.load_gather(cand_idx, [bslot])
            stage_i[pl.ds(j * 16, 16)] = blk * 128 + (pos & 127)

        pltpu.sync_copy(stage_v, vals_ref.at[r])
        pltpu.sync_copy(stage_i, gidx_ref.at[r])
        return 0

    lax.fori_loop(0, rpw, row_body, 0)


def kernel(query_embeddings, item_embeddings_t, item_ids, k):
    bq, d = query_embeddings.shape
    nx = item_embeddings_t.shape[1]
    rt = 32

    items_p = jnp.pad(item_embeddings_t, ((0, 0), (0, XP - nx)))
    mesh = plsc.VectorSubcoreMesh(core_axis_name="c", subcore_axis_name="s")
    ngrp = 4
    bg = bq // ngrp
    vparts, gparts = [], []
    for g in range(ngrp):
        qg = lax.slice_in_dim(query_embeddings, g * bg, (g + 1) * bg, axis=0)
        keys, bmax = pl.pallas_call(
            functools.partial(_tc_body, nx),
            grid=(bg // rt,),
            in_specs=[
                pl.BlockSpec((rt, d), lambda i: (i, 0)),
                pl.BlockSpec((d, XP), lambda i: (0, 0)),
            ],
            out_specs=[
                pl.BlockSpec((rt, XP), lambda i: (i, 0)),
                pl.BlockSpec((rt, NB), lambda i: (i, 0)),
            ],
            out_shape=[
                jax.ShapeDtypeStruct((bg, XP), jnp.uint32),
                jax.ShapeDtypeStruct((bg, NB), jnp.uint32),
            ],
        )(qg, items_p)
        keys2d = keys.reshape(bg * NB, 128)
        vg, gg = pl.kernel(
            functools.partial(_sc_topk, bg // NWORK),
            mesh=mesh,
            compiler_params=pltpu.CompilerParams(needs_layout_passes=False),
            out_type=[
                jax.ShapeDtypeStruct((bg, SEL), jnp.float32),
                jax.ShapeDtypeStruct((bg, SEL), jnp.int32),
            ],
            scratch_types=[
                pltpu.VMEM((CAND_ROWS, 128), jnp.uint32),
                pltpu.VMEM((NB,), jnp.uint32),
                pltpu.VMEM((CAND_ROWS,), jnp.int32),
                pltpu.VMEM((CAND_ROWS,), jnp.int32),
                pltpu.VMEM((4096,), jnp.int32),
                pltpu.VMEM((SEL,), jnp.int32),
                pltpu.VMEM((SEL,), jnp.int32),
                pltpu.VMEM((SEL,), jnp.float32),
                pltpu.VMEM((SEL,), jnp.int32),
                pltpu.SMEM((8,), jnp.int32),
                pltpu.SemaphoreType.DMA,
            ],
        )(keys2d, bmax)
        vparts.append(vg)
        gparts.append(gg)
    vals128 = jnp.concatenate(vparts, axis=0)
    gidx128 = jnp.concatenate(gparts, axis=0)

    topk_logits = vals128[:, :KTOP]
    topk_indices = gidx128[:, :KTOP]
    topk_logits = jnp.nan_to_num(topk_logits, nan=-1000000000.0,
                                 posinf=1000000000.0, neginf=-1000000000.0)
    topk_indices = topk_indices + jnp.asarray(k - KTOP, dtype=jnp.int32)
    topk_indices = jnp.clip(topk_indices, 0, nx - 1)
    topk_item_ids = jnp.take(item_ids[0], topk_indices, axis=0)
    return (topk_logits, topk_item_ids)


# speedup vs baseline: 2.0224x; 1.0008x over previous
import functools

import jax
import jax.numpy as jnp
from jax import lax
from jax.experimental import pallas as pl
from jax.experimental.pallas import tpu as pltpu
from jax.experimental.pallas import tpu_sc as plsc

XP = 100352
NB = XP // 128
KTOP = 100
NWORK = 32
CH = 64
CAND_ROWS = 832
SEL = 128


def _order_u32(x):
    u = lax.bitcast_convert_type(x, jnp.uint32)
    neg = u >= jnp.uint32(0x80000000)
    return jnp.where(neg, jnp.uint32(0) - u, u ^ jnp.uint32(0x80000000))


def _tc_body(nx, q_ref, it_ref, keys_ref, bm_ref):
    logits = jnp.dot(q_ref[...], it_ref[...],
                     preferred_element_type=jnp.float32)
    col = lax.broadcasted_iota(jnp.int32, logits.shape, 1)
    logits = jnp.where(col < nx, logits, jnp.float32(-1e30))
    keys_ref[...] = _order_u32(logits)
    r = logits.reshape(logits.shape[0], NB, 128)
    bm_ref[...] = _order_u32(jnp.max(r, axis=2))


def _sc_topk(rpw, keys2d, bm, vals_ref, gidx_ref,
             cand_buf, bm_v, cand_idx, cand_gid, hist, sel_o, sel_gi,
             stage_v, stage_i, smem, sem):
    nc = 2
    wid = lax.axis_index("s") * nc + lax.axis_index("c")
    lanes = lax.iota(jnp.int32, 16)
    lanes256 = lanes * 256
    ones = jnp.ones((16,), jnp.int32)

    def popc(mask):
        return plsc.all_reduce_population_count(mask)[0]

    def group_tot(g):
        t = hist[pl.ds(g * 16, 16)]
        for l in range(1, 16):
            t = t + hist[pl.ds(l * 256 + g * 16, 16)]
        return t

    def hist_scan(kth):
        def cond(c):
            g, cum, stop = c
            return jnp.logical_and(stop == 0, g >= 0)

        def body(c):
            g, cum, stop = c
            gsum = jnp.sum(group_tot(g))
            hit = (cum + gsum) >= kth
            return (jnp.where(hit, g, g - 1),
                    jnp.where(hit, cum, cum + gsum),
                    jnp.where(hit, 1, 0).astype(jnp.int32))

        g, cum, _ = lax.while_loop(cond, body, (jnp.int32(15),
                                                jnp.int32(0),
                                                jnp.int32(0)))
        g = jnp.maximum(g, 0)
        totv = group_tot(g)
        rsfx = plsc.cumsum(lax.rev(totv, (0,)))
        revmask = (cum + rsfx) >= kth
        f = plsc.all_reduce_ffs(revmask)
        f0 = f[0] if getattr(f, 'ndim', 0) else f
        i_star = 15 - f0
        onehot_r = lanes == f0
        onehot = lanes == i_star
        inb = jnp.sum(jnp.where(onehot, totv, 0))
        sstar = jnp.sum(jnp.where(onehot_r, rsfx, 0))
        return g * 16 + i_star, cum + sstar - inb, inb

    def zero_hist():
        zv = jnp.zeros((16,), jnp.int32)

        def zh(i, _):
            for s in range(8):
                hist[pl.ds(i * 128 + s * 16, 16)] = zv
            return 0
        lax.fori_loop(0, 32, zh, 0)

    def row_body(i, _):
        r = wid * rpw + i
        pltpu.sync_copy(bm.at[r], bm_v)

        zero_hist()

        def bpass0(j, _):
            o = bm_v[pl.ds(j * 16, 16)]
            d = lax.convert_element_type(o >> jnp.uint32(24), jnp.int32)
            plsc.addupdate_scatter(hist, [lanes256 + d], ones)
            return 0
        with jax.named_scope("ph_blk0"):
            lax.fori_loop(0, NB // 16, bpass0, 0)
            b0, ab0, _ = hist_scan(jnp.int32(KTOP))

        zero_hist()
        b0u = lax.convert_element_type(b0, jnp.uint32)

        def bpass1(j, _):
            o = bm_v[pl.ds(j * 16, 16)]
            m = (o >> jnp.uint32(24)) == b0u
            d = lax.convert_element_type(
                (o >> jnp.uint32(16)) & jnp.uint32(0xFF), jnp.int32)
            plsc.addupdate_scatter(hist, [lanes256 + d], ones, mask=m)
            return 0
        with jax.named_scope("ph_blk1"):
            lax.fori_loop(0, NB // 16, bpass1, 0)
            kth1 = jnp.int32(KTOP) - ab0
            b1, _, _ = hist_scan(jnp.maximum(kth1, 1))
        bp = (b0 * 256 + b1)
        bpu = lax.convert_element_type(bp, jnp.uint32)

        def pre(j, _):
            cand_gid[pl.ds(j * 16, 16)] = jnp.full((16,), r * NB, jnp.int32)
            return 0
        lax.fori_loop(0, CAND_ROWS // 16, pre, 0)

        def bcomp(j, cnt):
            o = bm_v[pl.ds(j * 16, 16)]
            keep = (o >> jnp.uint32(16)) >= bpu
            ids = j * 16 + lanes
            plsc.store_compressed(cand_idx.at[pl.ds(cnt, 16)], ids, mask=keep)
            plsc.store_compressed(cand_gid.at[pl.ds(cnt, 16)],
                                  ids + r * NB, mask=keep)
            return cnt + popc(keep)
        with jax.named_scope("ph_bcomp"):
            cnt = plsc.parallel_loop(0, NB // 16, carry=jnp.int32(0))(bcomp)

        nchunks = (cnt + CH - 1) // CH

        def gat(c, _):
            pltpu.async_copy(
                keys2d.at[cand_gid.at[pl.ds(c * CH, CH)]],
                cand_buf.at[pl.ds(c * CH, CH)], sem)
            return 0
        with jax.named_scope("ph_gat"):
            lax.fori_loop(0, nchunks, gat, 0)

        def drain(c, _):
            pltpu.make_async_copy(
                keys2d.at[cand_gid.at[pl.ds(c * CH, CH)]],
                cand_buf.at[pl.ds(c * CH, CH)], sem).wait()
            return 0
        with jax.named_scope("ph_drain"):
            lax.fori_loop(0, nchunks, drain, 0)

        smem[0] = 0
        smem[1] = jnp.where(ab0 == 0, b0, 0)
        smem[2] = 24
        smem[3] = 0
        smem[4] = 0
        smem[6] = jnp.where(ab0 == 0, 1, 0)

        for lev in range(4):
            sh = 24 - 8 * lev
            if lev == 0:
                run_lev = jnp.logical_and(smem[0] == 0, smem[6] == 0)
            else:
                run_lev = smem[0] == 0

            @pl.when(run_lev)
            def _level():
                zero_hist()
                pu = lax.convert_element_type(smem[1], jnp.uint32)

                def epass(row, _):
                    for sub in range(8):
                        o = cand_buf[row, pl.ds(sub * 16, 16)]
                        d = lax.convert_element_type(
                            (o >> jnp.uint32(sh)) & jnp.uint32(0xFF),
                            jnp.int32)
                        if lev == 0:
                            plsc.addupdate_scatter(
                                hist, [lanes256 + d], ones)
                        else:
                            m = (o >> jnp.uint32(sh + 8)) == pu
                            plsc.addupdate_scatter(
                                hist, [lanes256 + d], ones, mask=m)
                    return 0
                with jax.named_scope("ph_epass%d" % lev):
                    plsc.parallel_loop(0, cnt)(lambda row: epass(row, 0) and None)

                kneed = jnp.int32(KTOP) - smem[3]
                bb, above, inb = hist_scan(kneed)
                smem[1] = smem[1] * 256 + bb
                smem[2] = sh
                smem[3] = smem[3] + above
                fits = (smem[3] + inb) <= SEL

                @pl.when(fits)
                def _():
                    smem[0] = 1
                    smem[4] = 0
                if lev == 3:
                    @pl.when(jnp.logical_not(fits))
                    def _():
                        smem[0] = 1
                        smem[4] = 1

        def selpre(j, _):
            sel_o[pl.ds(j * 16, 16)] = jnp.zeros((16,), jnp.int32)
            sel_gi[pl.ds(j * 16, 16)] = (
                jnp.int32(0x7FFF0000) + j * 16 + lanes)
            return 0
        lax.fori_loop(0, SEL // 16, selpre, 0)

        pfin = lax.convert_element_type(smem[1], jnp.uint32)
        shfin = lax.convert_element_type(smem[2], jnp.uint32)

        @pl.when(smem[4] == 0)
        def _rank_compact():
            def cpass(row, ns):
                for sub in range(8):
                    o = cand_buf[row, pl.ds(sub * 16, 16)]
                    keep = (o >> shfin) >= pfin
                    pos = row * 128 + (sub * 16 + lanes)
                    plsc.store_compressed(
                        sel_o.at[pl.ds(ns, 16)],
                        plsc.bitcast(o, jnp.int32), mask=keep)
                    plsc.store_compressed(sel_gi.at[pl.ds(ns, 16)],
                                          pos, mask=keep)
                    ns = ns + popc(keep)
                return ns
            with jax.named_scope("ph_rcomp"):
                plsc.parallel_loop(0, cnt, carry=jnp.int32(0))(cpass)

        @pl.when(smem[4] == 1)
        def _equals_compact():
            need = jnp.int32(KTOP) - smem[3]

            def cpass(row, c):
                ns, ne = c
                for sub in range(8):
                    o = cand_buf[row, pl.ds(sub * 16, 16)]
                    pos = row * 128 + (sub * 16 + lanes)
                    keep = o > pfin
                    plsc.store_compressed(
                        sel_o.at[pl.ds(ns, 16)],
                        plsc.bitcast(o, jnp.int32), mask=keep)
                    plsc.store_compressed(sel_gi.at[pl.ds(ns, 16)],
                                          pos, mask=keep)
                    ns = ns + popc(keep)
                    eq = o == pfin
                    pref = plsc.cumsum(jnp.where(eq, 1, 0))
                    wm = jnp.logical_and(eq, (ne + pref) <= need)
                    plsc.store_compressed(
                        sel_o.at[pl.ds(ns, 16)],
                        plsc.bitcast(o, jnp.int32), mask=wm)
                    plsc.store_compressed(sel_gi.at[pl.ds(ns, 16)],
                                          pos, mask=wm)
                    npc = popc(wm)
                    ns = ns + npc
                    ne = ne + npc
                return (ns, ne)
            lax.fori_loop(0, cnt, cpass,
                          (smem[3] + jnp.int32(0), jnp.int32(0)))

        def rank_i(ii, _):
            iv = jnp.full((16,), ii, jnp.int32)
            oib = plsc.bitcast(plsc.load_gather(sel_o, [iv]), jnp.uint32)
            gib = plsc.load_gather(sel_gi, [iv])

            rank = jnp.int32(0)
            for j in range(SEL // 16):
                o = plsc.bitcast(sel_o[pl.ds(j * 16, 16)], jnp.uint32)
                g = sel_gi[pl.ds(j * 16, 16)]
                beat = jnp.logical_or(
                    o > oib, jnp.logical_and(o == oib, g < gib))
                rank = rank + popc(beat)
            u = jnp.where(oib >= jnp.uint32(0x80000000),
                          oib ^ jnp.uint32(0x80000000), jnp.uint32(0) - oib)
            val_v = plsc.bitcast(u, jnp.float32)
            rv = jnp.full((16,), rank, jnp.int32)
            lm = lanes == 0
            plsc.store_scatter(stage_v, [rv], val_v, mask=lm)
            plsc.store_scatter(stage_i, [rv], gib, mask=lm)
            return 0
        with jax.named_scope("ph_rank"):
            plsc.parallel_loop(0, SEL)(lambda ii: rank_i(ii, 0) and None)

        for j in range(SEL // 16):
            pos = stage_i[pl.ds(j * 16, 16)]
            bslot = jnp.minimum(pos >> 7, jnp.int32(CAND_ROWS - 1))
            blk = plsc.load_gather(cand_idx, [bslot])
            stage_i[pl.ds(j * 16, 16)] = blk * 128 + (pos & 127)

        pltpu.sync_copy(stage_v, vals_ref.at[r])
        pltpu.sync_copy(stage_i, gidx_ref.at[r])
        return 0

    lax.fori_loop(0, rpw, row_body, 0)


def kernel(query_embeddings, item_embeddings_t, item_ids, k):
    bq, d = query_embeddings.shape
    nx = item_embeddings_t.shape[1]
    rt = 32

    items_p = jnp.pad(item_embeddings_t, ((0, 0), (0, XP - nx)))
    mesh = plsc.VectorSubcoreMesh(core_axis_name="c", subcore_axis_name="s")
    ngrp = 8
    bg = bq // ngrp
    vparts, gparts = [], []
    for g in range(ngrp):
        qg = lax.slice_in_dim(query_embeddings, g * bg, (g + 1) * bg, axis=0)
        keys, bmax = pl.pallas_call(
            functools.partial(_tc_body, nx),
            grid=(bg // rt,),
            in_specs=[
                pl.BlockSpec((rt, d), lambda i: (i, 0)),
                pl.BlockSpec((d, XP), lambda i: (0, 0)),
            ],
            out_specs=[
                pl.BlockSpec((rt, XP), lambda i: (i, 0)),
                pl.BlockSpec((rt, NB), lambda i: (i, 0)),
            ],
            out_shape=[
                jax.ShapeDtypeStruct((bg, XP), jnp.uint32),
                jax.ShapeDtypeStruct((bg, NB), jnp.uint32),
            ],
        )(qg, items_p)
        keys2d = keys.reshape(bg * NB, 128)
        vg, gg = pl.kernel(
            functools.partial(_sc_topk, bg // NWORK),
            mesh=mesh,
            compiler_params=pltpu.CompilerParams(needs_layout_passes=False),
            out_type=[
                jax.ShapeDtypeStruct((bg, SEL), jnp.float32),
                jax.ShapeDtypeStruct((bg, SEL), jnp.int32),
            ],
            scratch_types=[
                pltpu.VMEM((CAND_ROWS, 128), jnp.uint32),
                pltpu.VMEM((NB,), jnp.uint32),
                pltpu.VMEM((CAND_ROWS,), jnp.int32),
                pltpu.VMEM((CAND_ROWS,), jnp.int32),
                pltpu.VMEM((4096,), jnp.int32),
                pltpu.VMEM((SEL,), jnp.int32),
                pltpu.VMEM((SEL,), jnp.int32),
                pltpu.VMEM((SEL,), jnp.float32),
                pltpu.VMEM((SEL,), jnp.int32),
                pltpu.SMEM((8,), jnp.int32),
                pltpu.SemaphoreType.DMA,
            ],
        )(keys2d, bmax)
        vparts.append(vg)
        gparts.append(gg)
    vals128 = jnp.concatenate(vparts, axis=0)
    gidx128 = jnp.concatenate(gparts, axis=0)

    topk_logits = vals128[:, :KTOP]
    topk_indices = gidx128[:, :KTOP]
    topk_logits = jnp.nan_to_num(topk_logits, nan=-1000000000.0,
                                 posinf=1000000000.0, neginf=-1000000000.0)
    topk_indices = topk_indices + jnp.asarray(k - KTOP, dtype=jnp.int32)
    topk_indices = jnp.clip(topk_indices, 0, nx - 1)
    topk_item_ids = jnp.take(item_ids[0], topk_indices, axis=0)
    return (topk_logits, topk_item_ids)


# speedup vs baseline: 2.2728x; 1.1238x over previous
import functools

import jax
import jax.numpy as jnp
from jax import lax
from jax.experimental import pallas as pl
from jax.experimental.pallas import tpu as pltpu
from jax.experimental.pallas import tpu_sc as plsc

XP = 100352
NB = XP // 128
KTOP = 100
NWORK = 32
CH = 64
CAND_ROWS = 832
SEL = 128


def _order_u32(x):
    u = lax.bitcast_convert_type(x, jnp.uint32)
    neg = u >= jnp.uint32(0x80000000)
    return jnp.where(neg, jnp.uint32(0) - u, u ^ jnp.uint32(0x80000000))


def _tc_body(nx, q_ref, it_ref, keys_ref, bm_ref):
    logits = jnp.dot(q_ref[...], it_ref[...],
                     preferred_element_type=jnp.float32)
    col = lax.broadcasted_iota(jnp.int32, logits.shape, 1)
    logits = jnp.where(col < nx, logits, jnp.float32(-1e30))
    keys_ref[...] = _order_u32(logits)
    r = logits.reshape(logits.shape[0], NB, 128)
    bm_ref[...] = _order_u32(jnp.max(r, axis=2))


def _sc_topk(rpw, keys2d, bm, vals_ref, gidx_ref,
             cand_buf, bm_v, cand_idx, cand_gid, hist, sel_o, sel_gi,
             stage_v, stage_i, smem, soff, sem):
    nc = 2
    wid = lax.axis_index("s") * nc + lax.axis_index("c")
    lanes = lax.iota(jnp.int32, 16)
    lanes256 = lanes * 256
    ones = jnp.ones((16,), jnp.int32)

    def popc(mask):
        return plsc.all_reduce_population_count(mask)[0]

    def group_tot(g):
        t = hist[pl.ds(g * 16, 16)]
        for l in range(1, 16):
            t = t + hist[pl.ds(l * 256 + g * 16, 16)]
        return t

    def hist_scan(kth):
        def cond(c):
            g, cum, stop = c
            return jnp.logical_and(stop == 0, g >= 0)

        def body(c):
            g, cum, stop = c
            gsum = jnp.sum(group_tot(g))
            hit = (cum + gsum) >= kth
            return (jnp.where(hit, g, g - 1),
                    jnp.where(hit, cum, cum + gsum),
                    jnp.where(hit, 1, 0).astype(jnp.int32))

        g, cum, _ = lax.while_loop(cond, body, (jnp.int32(15),
                                                jnp.int32(0),
                                                jnp.int32(0)))
        g = jnp.maximum(g, 0)
        totv = group_tot(g)
        rsfx = plsc.cumsum(lax.rev(totv, (0,)))
        revmask = (cum + rsfx) >= kth
        f = plsc.all_reduce_ffs(revmask)
        f0 = f[0] if getattr(f, 'ndim', 0) else f
        i_star = 15 - f0
        onehot_r = lanes == f0
        onehot = lanes == i_star
        inb = jnp.sum(jnp.where(onehot, totv, 0))
        sstar = jnp.sum(jnp.where(onehot_r, rsfx, 0))
        return g * 16 + i_star, cum + sstar - inb, inb

    def zero_hist():
        zv = jnp.zeros((16,), jnp.int32)

        def zh(i, _):
            for s in range(8):
                hist[pl.ds(i * 128 + s * 16, 16)] = zv
            return 0
        lax.fori_loop(0, 32, zh, 0)

    def row_body(i, _):
        r = wid * rpw + i
        pltpu.sync_copy(bm.at[r], bm_v)

        zero_hist()

        def bpass0(j, _):
            o = bm_v[pl.ds(j * 16, 16)]
            d = lax.convert_element_type(o >> jnp.uint32(24), jnp.int32)
            plsc.addupdate_scatter(hist, [lanes256 + d], ones)
            return 0
        with jax.named_scope("ph_blk0"):
            lax.fori_loop(0, NB // 16, bpass0, 0)
            b0, ab0, _ = hist_scan(jnp.int32(KTOP))

        zero_hist()
        b0u = lax.convert_element_type(b0, jnp.uint32)

        def bpass1(j, _):
            o = bm_v[pl.ds(j * 16, 16)]
            m = (o >> jnp.uint32(24)) == b0u
            d = lax.convert_element_type(
                (o >> jnp.uint32(16)) & jnp.uint32(0xFF), jnp.int32)
            plsc.addupdate_scatter(hist, [lanes256 + d], ones, mask=m)
            return 0
        with jax.named_scope("ph_blk1"):
            lax.fori_loop(0, NB // 16, bpass1, 0)
            kth1 = jnp.int32(KTOP) - ab0
            b1, _, _ = hist_scan(jnp.maximum(kth1, 1))
        bp = (b0 * 256 + b1)
        bpu = lax.convert_element_type(bp, jnp.uint32)

        def pre(j, _):
            cand_gid[pl.ds(j * 16, 16)] = jnp.full((16,), r * NB, jnp.int32)
            return 0
        lax.fori_loop(0, CAND_ROWS // 16, pre, 0)

        def bcomp(j, cnt):
            o = bm_v[pl.ds(j * 16, 16)]
            keep = (o >> jnp.uint32(16)) >= bpu
            ids = j * 16 + lanes
            plsc.store_compressed(cand_idx.at[pl.ds(cnt, 16)], ids, mask=keep)
            plsc.store_compressed(cand_gid.at[pl.ds(cnt, 16)],
                                  ids + r * NB, mask=keep)
            return cnt + popc(keep)
        with jax.named_scope("ph_bcomp"):
            cnt = plsc.parallel_loop(0, NB // 16, carry=jnp.int32(0))(bcomp)

        nchunks = (cnt + CH - 1) // CH

        def gat(c, _):
            pltpu.async_copy(
                keys2d.at[cand_gid.at[pl.ds(c * CH, CH)]],
                cand_buf.at[pl.ds(c * CH, CH)], sem)
            return 0
        with jax.named_scope("ph_gat"):
            lax.fori_loop(0, nchunks, gat, 0)

        def drain(c, _):
            pltpu.make_async_copy(
                keys2d.at[cand_gid.at[pl.ds(c * CH, CH)]],
                cand_buf.at[pl.ds(c * CH, CH)], sem).wait()
            return 0
        with jax.named_scope("ph_drain"):
            lax.fori_loop(0, nchunks, drain, 0)

        smem[0] = 0
        smem[1] = jnp.where(ab0 == 0, b0, 0)
        smem[2] = 24
        smem[3] = 0
        smem[4] = 0
        smem[6] = jnp.where(ab0 == 0, 1, 0)

        for lev in range(4):
            sh = 24 - 8 * lev
            if lev == 0:
                run_lev = jnp.logical_and(smem[0] == 0, smem[6] == 0)
            else:
                run_lev = smem[0] == 0

            @pl.when(run_lev)
            def _level():
                zero_hist()
                pu = lax.convert_element_type(smem[1], jnp.uint32)

                def epass(row, _):
                    for sub in range(8):
                        o = cand_buf[row, pl.ds(sub * 16, 16)]
                        d = lax.convert_element_type(
                            (o >> jnp.uint32(sh)) & jnp.uint32(0xFF),
                            jnp.int32)
                        if lev == 0:
                            plsc.addupdate_scatter(
                                hist, [lanes256 + d], ones)
                        else:
                            m = (o >> jnp.uint32(sh + 8)) == pu
                            plsc.addupdate_scatter(
                                hist, [lanes256 + d], ones, mask=m)
                    return 0
                with jax.named_scope("ph_epass%d" % lev):
                    plsc.parallel_loop(0, cnt)(lambda row: epass(row, 0) and None)

                kneed = jnp.int32(KTOP) - smem[3]
                bb, above, inb = hist_scan(kneed)
                smem[1] = smem[1] * 256 + bb
                smem[2] = sh
                smem[3] = smem[3] + above
                fits = (smem[3] + inb) <= SEL

                @pl.when(fits)
                def _():
                    smem[0] = 1
                    smem[4] = 0
                if lev == 3:
                    @pl.when(jnp.logical_not(fits))
                    def _():
                        smem[0] = 1
                        smem[4] = 1

        def selpre(j, _):
            sel_o[pl.ds(j * 16, 16)] = jnp.zeros((16,), jnp.int32)
            sel_gi[pl.ds(j * 16, 16)] = (
                jnp.int32(0x7FFF0000) + j * 16 + lanes)
            return 0
        lax.fori_loop(0, SEL // 16, selpre, 0)

        pfin = lax.convert_element_type(smem[1], jnp.uint32)
        shfin = lax.convert_element_type(smem[2], jnp.uint32)

        @pl.when(smem[4] == 0)
        def _rank_compact():
            def c1(row, ns):
                soff[row] = ns
                pc = jnp.int32(0)
                for sub in range(8):
                    o = cand_buf[row, pl.ds(sub * 16, 16)]
                    pc = pc + popc((o >> shfin) >= pfin)
                return ns + pc

            def c2(row):
                ns = soff[row]
                for sub in range(8):
                    o = cand_buf[row, pl.ds(sub * 16, 16)]
                    keep = (o >> shfin) >= pfin
                    pos = row * 128 + (sub * 16 + lanes)
                    plsc.store_compressed(
                        sel_o.at[pl.ds(ns, 16)],
                        plsc.bitcast(o, jnp.int32), mask=keep)
                    plsc.store_compressed(sel_gi.at[pl.ds(ns, 16)],
                                          pos, mask=keep)
                    ns = ns + popc(keep)

            with jax.named_scope("ph_rcomp"):
                plsc.parallel_loop(0, cnt, carry=jnp.int32(0))(c1)
                plsc.parallel_loop(0, cnt)(c2)

        @pl.when(smem[4] == 1)
        def _equals_compact():
            need = jnp.int32(KTOP) - smem[3]

            def cpass(row, c):
                ns, ne = c
                for sub in range(8):
                    o = cand_buf[row, pl.ds(sub * 16, 16)]
                    pos = row * 128 + (sub * 16 + lanes)
                    keep = o > pfin
                    plsc.store_compressed(
                        sel_o.at[pl.ds(ns, 16)],
                        plsc.bitcast(o, jnp.int32), mask=keep)
                    plsc.store_compressed(sel_gi.at[pl.ds(ns, 16)],
                                          pos, mask=keep)
                    ns = ns + popc(keep)
                    eq = o == pfin
                    pref = plsc.cumsum(jnp.where(eq, 1, 0))
                    wm = jnp.logical_and(eq, (ne + pref) <= need)
                    plsc.store_compressed(
                        sel_o.at[pl.ds(ns, 16)],
                        plsc.bitcast(o, jnp.int32), mask=wm)
                    plsc.store_compressed(sel_gi.at[pl.ds(ns, 16)],
                                          pos, mask=wm)
                    npc = popc(wm)
                    ns = ns + npc
                    ne = ne + npc
                return (ns, ne)
            lax.fori_loop(0, cnt, cpass,
                          (smem[3] + jnp.int32(0), jnp.int32(0)))

        def rank_i(ii, _):
            iv = jnp.full((16,), ii, jnp.int32)
            oib = plsc.bitcast(plsc.load_gather(sel_o, [iv]), jnp.uint32)
            gib = plsc.load_gather(sel_gi, [iv])

            rank = jnp.int32(0)
            for j in range(SEL // 16):
                o = plsc.bitcast(sel_o[pl.ds(j * 16, 16)], jnp.uint32)
                g = sel_gi[pl.ds(j * 16, 16)]
                beat = jnp.logical_or(
                    o > oib, jnp.logical_and(o == oib, g < gib))
                rank = rank + popc(beat)
            u = jnp.where(oib >= jnp.uint32(0x80000000),
                          oib ^ jnp.uint32(0x80000000), jnp.uint32(0) - oib)
            val_v = plsc.bitcast(u, jnp.float32)
            rv = jnp.full((16,), rank, jnp.int32)
            lm = lanes == 0
            plsc.store_scatter(stage_v, [rv], val_v, mask=lm)
            plsc.store_scatter(stage_i, [rv], gib, mask=lm)
            return 0
        with jax.named_scope("ph_rank"):
            plsc.parallel_loop(0, SEL)(lambda ii: rank_i(ii, 0) and None)

        for j in range(SEL // 16):
            pos = stage_i[pl.ds(j * 16, 16)]
            bslot = jnp.minimum(pos >> 7, jnp.int32(CAND_ROWS - 1))
            blk = plsc.load_gather(cand_idx, [bslot])
            stage_i[pl.ds(j * 16, 16)] = blk * 128 + (pos & 127)

        pltpu.sync_copy(stage_v, vals_ref.at[r])
        pltpu.sync_copy(stage_i, gidx_ref.at[r])
        return 0

    lax.fori_loop(0, rpw, row_body, 0)


def kernel(query_embeddings, item_embeddings_t, item_ids, k):
    bq, d = query_embeddings.shape
    nx = item_embeddings_t.shape[1]
    rt = 32

    items_p = jnp.pad(item_embeddings_t, ((0, 0), (0, XP - nx)))
    mesh = plsc.VectorSubcoreMesh(core_axis_name="c", subcore_axis_name="s")
    ngrp = 4
    bg = bq // ngrp
    vparts, gparts = [], []
    for g in range(ngrp):
        qg = lax.slice_in_dim(query_embeddings, g * bg, (g + 1) * bg, axis=0)
        keys, bmax = pl.pallas_call(
            functools.partial(_tc_body, nx),
            grid=(bg // rt,),
            in_specs=[
                pl.BlockSpec((rt, d), lambda i: (i, 0)),
                pl.BlockSpec((d, XP), lambda i: (0, 0)),
            ],
            out_specs=[
                pl.BlockSpec((rt, XP), lambda i: (i, 0)),
                pl.BlockSpec((rt, NB), lambda i: (i, 0)),
            ],
            out_shape=[
                jax.ShapeDtypeStruct((bg, XP), jnp.uint32),
                jax.ShapeDtypeStruct((bg, NB), jnp.uint32),
            ],
        )(qg, items_p)
        keys2d = keys.reshape(bg * NB, 128)
        vg, gg = pl.kernel(
            functools.partial(_sc_topk, bg // NWORK),
            mesh=mesh,
            compiler_params=pltpu.CompilerParams(needs_layout_passes=False),
            out_type=[
                jax.ShapeDtypeStruct((bg, SEL), jnp.float32),
                jax.ShapeDtypeStruct((bg, SEL), jnp.int32),
            ],
            scratch_types=[
                pltpu.VMEM((CAND_ROWS, 128), jnp.uint32),
                pltpu.VMEM((NB,), jnp.uint32),
                pltpu.VMEM((CAND_ROWS,), jnp.int32),
                pltpu.VMEM((CAND_ROWS,), jnp.int32),
                pltpu.VMEM((4096,), jnp.int32),
                pltpu.VMEM((SEL,), jnp.int32),
                pltpu.VMEM((SEL,), jnp.int32),
                pltpu.VMEM((SEL,), jnp.float32),
                pltpu.VMEM((SEL,), jnp.int32),
                pltpu.SMEM((8,), jnp.int32),
                pltpu.SMEM((CAND_ROWS,), jnp.int32),
                pltpu.SemaphoreType.DMA,
            ],
        )(keys2d, bmax)
        vparts.append(vg)
        gparts.append(gg)
    vals128 = jnp.concatenate(vparts, axis=0)
    gidx128 = jnp.concatenate(gparts, axis=0)

    topk_logits = vals128[:, :KTOP]
    topk_indices = gidx128[:, :KTOP]
    topk_logits = jnp.nan_to_num(topk_logits, nan=-1000000000.0,
                                 posinf=1000000000.0, neginf=-1000000000.0)
    topk_indices = topk_indices + jnp.asarray(k - KTOP, dtype=jnp.int32)
    topk_indices = jnp.clip(topk_indices, 0, nx - 1)
    topk_item_ids = jnp.take(item_ids[0], topk_indices, axis=0)
    return (topk_logits, topk_item_ids)
